# baseline (speedup 1.0000x reference)
import functools

import jax
import jax.numpy as jnp
from jax import lax
from jax.experimental import pallas as pl
from jax.experimental.pallas import tpu as pltpu
from jax.experimental.pallas import tpu_sc as plsc


_SB = 80
_NW = 32
_BR = 200


@functools.lru_cache(maxsize=None)
def _make_scatter(N: int, M: int, D: int):
  assert M == 5000 and D % 16 == 0
  n_full = M // _SB
  tail = M - n_full * _SB
  tail_off = n_full * _SB

  mesh = plsc.VectorSubcoreMesh(core_axis_name="c", subcore_axis_name="s")

  @functools.partial(
      pl.kernel,
      mesh=mesh,
      out_type=(),
      scratch_types=[
          pltpu.VMEM((_SB,), jnp.int32),
          pltpu.VMEM((_SB,), jnp.int32),
          pltpu.VMEM((2 * _SB, D), jnp.float32),
          pltpu.VMEM((tail,), jnp.int32),
          pltpu.VMEM((tail, D), jnp.float32),
          pltpu.SemaphoreType.DMA,
          pltpu.SemaphoreType.DMA,
      ],
  )
  def scatter(x_hbm, idx_hbm, out_hbm,
              idx_a, idx_b, x_v, idx_t, x_t, lsem, ssem):
    wid = lax.axis_index("c") * 16 + lax.axis_index("s")
    b0 = wid * 2
    b1 = wid * 2 + 1

    @pl.when(b0 < n_full)
    def _():
      pltpu.async_copy(idx_hbm.at[pl.ds(b0 * _SB, _SB)], idx_a, lsem)
      pltpu.async_copy(idx_hbm.at[pl.ds(b1 * _SB, _SB)], idx_b, lsem)
      pltpu.async_copy(x_hbm.at[pl.ds(b0 * _SB, 2 * _SB)], x_v, lsem)
      pltpu.make_async_copy(
          idx_hbm.at[pl.ds(b0 * _SB, _SB)], idx_a, lsem).wait()
      pltpu.make_async_copy(
          idx_hbm.at[pl.ds(b1 * _SB, _SB)], idx_b, lsem).wait()
      pltpu.make_async_copy(
          x_hbm.at[pl.ds(b0 * _SB, 2 * _SB)], x_v, lsem).wait()
      pltpu.async_copy(x_v.at[pl.ds(0, _SB)], out_hbm.at[idx_a], ssem)
      pltpu.async_copy(x_v.at[pl.ds(_SB, _SB)], out_hbm.at[idx_b], ssem)
      pltpu.make_async_copy(
          x_v.at[pl.ds(0, _SB)], out_hbm.at[idx_a], ssem).wait()
      pltpu.make_async_copy(
          x_v.at[pl.ds(_SB, _SB)], out_hbm.at[idx_b], ssem).wait()

    @pl.when(wid == _NW - 1)
    def _():
      pltpu.async_copy(idx_hbm.at[pl.ds(tail_off, tail)], idx_t, lsem)
      pltpu.async_copy(x_hbm.at[pl.ds(tail_off, tail)], x_t, lsem)
      pltpu.make_async_copy(
          idx_hbm.at[pl.ds(tail_off, tail)], idx_t, lsem).wait()
      pltpu.make_async_copy(
          x_hbm.at[pl.ds(tail_off, tail)], x_t, lsem).wait()
      pltpu.async_copy(x_t, out_hbm.at[idx_t], ssem).wait()

  return scatter


@functools.lru_cache(maxsize=None)
def _make_copy(N: int, K: int):
  assert N % _BR == 0 and _BR % 8 == 0
  grid = N // _BR

  def body(a_ref, aout_ref):
    aout_ref[...] = a_ref[...]

  return pl.pallas_call(
      body,
      grid=(grid,),
      in_specs=[pl.BlockSpec((_BR, K), lambda i: (i, 0))],
      out_specs=pl.BlockSpec((_BR, K), lambda i: (i, 0)),
      out_shape=jax.ShapeDtypeStruct((N, K), jnp.float32),
  )


@functools.lru_cache(maxsize=None)
def _make_zero(N: int, D: int):
  zr = 1000

  def body(z_ref):
    z_ref[...] = jnp.zeros_like(z_ref)

  return pl.pallas_call(
      body,
      grid=(N // zr,),
      out_specs=pl.BlockSpec((zr, D), lambda i: (i, 0)),
      out_shape=jax.ShapeDtypeStruct((N, D), jnp.float32),
  )


def kernel(A, X, idx):
  M, D = X.shape
  N = A.shape[0]
  z = _make_zero(N, D)()
  zref = jax.new_ref(z)
  _make_scatter(N, M, D)(X, idx.astype(jnp.int32), zref)
  A_out = _make_copy(N, A.shape[1])(A)
  return (A_out, zref[...])


# speedup vs baseline: 1.0064x; 1.0064x over previous
import functools

import jax
import jax.numpy as jnp
from jax import lax
from jax.experimental import pallas as pl
from jax.experimental.pallas import tpu as pltpu
from jax.experimental.pallas import tpu_sc as plsc


_SB = 80
_NW = 32
_BR = 200


@functools.lru_cache(maxsize=None)
def _make_scatter(N: int, M: int, D: int):
  assert M == 5000 and D % 16 == 0
  n_full = M // _SB
  tail = M - n_full * _SB
  tail_off = n_full * _SB

  mesh = plsc.VectorSubcoreMesh(core_axis_name="c", subcore_axis_name="s")

  @functools.partial(
      pl.kernel,
      mesh=mesh,
      out_type=(),
      scratch_types=[
          pltpu.VMEM((_SB,), jnp.int32),
          pltpu.VMEM((_SB,), jnp.int32),
          pltpu.VMEM((2 * _SB, D), jnp.float32),
          pltpu.VMEM((tail,), jnp.int32),
          pltpu.VMEM((tail, D), jnp.float32),
          pltpu.SemaphoreType.DMA,
          pltpu.SemaphoreType.DMA,
      ],
  )
  def scatter(x_hbm, idx_hbm, out_hbm,
              idx_a, idx_b, x_v, idx_t, x_t, lsem, ssem):
    wid = lax.axis_index("c") * 16 + lax.axis_index("s")
    b0 = wid * 2
    b1 = wid * 2 + 1

    @pl.when(b0 < n_full)
    def _():
      pltpu.async_copy(idx_hbm.at[pl.ds(b0 * _SB, _SB)], idx_a, lsem)
      pltpu.async_copy(idx_hbm.at[pl.ds(b1 * _SB, _SB)], idx_b, lsem)
      pltpu.async_copy(x_hbm.at[pl.ds(b0 * _SB, 2 * _SB)], x_v, lsem)
      pltpu.make_async_copy(
          idx_hbm.at[pl.ds(b0 * _SB, _SB)], idx_a, lsem).wait()
      pltpu.make_async_copy(
          idx_hbm.at[pl.ds(b1 * _SB, _SB)], idx_b, lsem).wait()
      pltpu.make_async_copy(
          x_hbm.at[pl.ds(b0 * _SB, 2 * _SB)], x_v, lsem).wait()
      pltpu.async_copy(x_v.at[pl.ds(0, _SB)], out_hbm.at[idx_a], ssem)
      pltpu.async_copy(x_v.at[pl.ds(_SB, _SB)], out_hbm.at[idx_b], ssem)
      pltpu.make_async_copy(
          x_v.at[pl.ds(0, _SB)], out_hbm.at[idx_a], ssem).wait()
      pltpu.make_async_copy(
          x_v.at[pl.ds(_SB, _SB)], out_hbm.at[idx_b], ssem).wait()

    @pl.when(wid == _NW - 1)
    def _():
      pltpu.async_copy(idx_hbm.at[pl.ds(tail_off, tail)], idx_t, lsem)
      pltpu.async_copy(x_hbm.at[pl.ds(tail_off, tail)], x_t, lsem)
      pltpu.make_async_copy(
          idx_hbm.at[pl.ds(tail_off, tail)], idx_t, lsem).wait()
      pltpu.make_async_copy(
          x_hbm.at[pl.ds(tail_off, tail)], x_t, lsem).wait()
      pltpu.async_copy(x_t, out_hbm.at[idx_t], ssem).wait()

  return scatter


@functools.lru_cache(maxsize=None)
def _make_copy(N: int, K: int):
  assert N % _BR == 0 and _BR % 8 == 0
  grid = N // _BR

  def body(a_ref, aout_ref):
    aout_ref[...] = a_ref[...]

  return pl.pallas_call(
      body,
      grid=(grid,),
      in_specs=[pl.BlockSpec((_BR, K), lambda i: (i, 0))],
      out_specs=pl.BlockSpec((_BR, K), lambda i: (i, 0)),
      out_shape=jax.ShapeDtypeStruct((N, K), jnp.float32),
  )


@functools.lru_cache(maxsize=None)
def _make_zero(N: int, D: int):
  zr = 1000

  def body(z_ref):
    z_ref[...] = jnp.zeros_like(z_ref)

  return pl.pallas_call(
      body,
      grid=(N // zr,),
      out_specs=pl.BlockSpec((zr, D), lambda i: (i, 0)),
      out_shape=jax.ShapeDtypeStruct((N, D), jnp.float32),
  )


def kernel(A, X, idx):
  M, D = X.shape
  N = A.shape[0]
  zref = jax.new_ref(jnp.zeros((N, D), jnp.float32))
  _make_scatter(N, M, D)(X, idx.astype(jnp.int32), zref)
  A_out = _make_copy(N, A.shape[1])(A)
  return (A_out, zref[...])
